# Initial kernel scaffold; baseline (speedup 1.0000x reference)
import functools

import jax
import jax.numpy as jnp
from jax import lax
from jax.experimental import pallas as pl
from jax.experimental.pallas import tpu as pltpu
from jax.experimental.pallas import tpu_sc as plsc

F32 = jnp.float32

N = 10000
E = 320000
D = 128
B = 16

BLK_E = 2560
BLK_N = 2000

NC = 2
NS = 16
NW = NC * NS
TPE = E // NW
CHUNK = 400
SUB = 80
NSUB = CHUNK // SUB
NIT = TPE // CHUNK
RPT = N // NS


def _dot(a, b):
    return jnp.dot(a, b, preferred_element_type=F32)


def _dot_t(a, b):
    return lax.dot_general(a, b, (((0,), (0,)), ((), ())),
                           preferred_element_type=F32)


def _pre_body(x_ref, bh_ref, u_ref, w1s_ref, w1d_ref, w1u_ref, b1_ref,
              xs2_ref, xd_ref):
    uu = _dot(u_ref[...], w1u_ref[...]) + b1_ref[...]
    xs2_ref[...] = _dot(x_ref[...], w1s_ref[...]) + _dot(bh_ref[...], uu)
    xd_ref[...] = _dot(x_ref[...], w1d_ref[...])


@jax.jit
def _tc_pre(x, bh, u, w1s, w1d, w1u, b1):
    nb = N // BLK_N
    return pl.pallas_call(
        _pre_body,
        grid=(nb,),
        in_specs=[
            pl.BlockSpec((BLK_N, D), lambda i: (i, 0)),
            pl.BlockSpec((BLK_N, B), lambda i: (i, 0)),
            pl.BlockSpec((B, D), lambda i: (0, 0)),
            pl.BlockSpec((D, D), lambda i: (0, 0)),
            pl.BlockSpec((D, D), lambda i: (0, 0)),
            pl.BlockSpec((D, D), lambda i: (0, 0)),
            pl.BlockSpec((1, D), lambda i: (0, 0)),
        ],
        out_specs=[
            pl.BlockSpec((BLK_N, D), lambda i: (i, 0)),
            pl.BlockSpec((BLK_N, D), lambda i: (i, 0)),
        ],
        out_shape=[
            jax.ShapeDtypeStruct((N, D), F32),
            jax.ShapeDtypeStruct((N, D), F32),
        ],
    )(x, bh, u, w1s, w1d, w1u, b1)


def _sc_gather_body(xs2, xd, bht, row, col,
                    gsa, gsb, bhe,
                    idxr, idxc, bufa, bufb, bufh, sem):
    wid = lax.axis_index("c") * NS + lax.axis_index("s")
    base = wid * TPE

    def body(i, _):
        off = pl.multiple_of(base + i * CHUNK, 8)
        pltpu.sync_copy(row.at[pl.ds(off, CHUNK)], idxr)
        pltpu.sync_copy(col.at[pl.ds(off, CHUNK)], idxc)
        cps = []
        for k in range(NSUB):
            s = pl.ds(k * SUB, SUB)
            cps.append(pltpu.async_copy(xs2.at[idxr.at[s]], bufa.at[s], sem))
            cps.append(pltpu.async_copy(xd.at[idxc.at[s]], bufb.at[s], sem))
            cps.append(pltpu.async_copy(bht.at[idxr.at[s]], bufh.at[s], sem))
        for cp in cps:
            cp.wait()
        pltpu.sync_copy(bufa, gsa.at[pl.ds(off, CHUNK)])
        pltpu.sync_copy(bufb, gsb.at[pl.ds(off, CHUNK)])
        pltpu.sync_copy(bufh, bhe.at[pl.ds(off, CHUNK)])
        return 0

    lax.fori_loop(0, NIT, body, 0)


@jax.jit
def _sc_gather(xs2, xd, bht, row, col):
    f = pl.kernel(
        _sc_gather_body,
        mesh=plsc.VectorSubcoreMesh(core_axis_name="c", subcore_axis_name="s"),
        out_type=[
            jax.ShapeDtypeStruct((E, D), F32),
            jax.ShapeDtypeStruct((E, D), F32),
            jax.ShapeDtypeStruct((E, B), F32),
        ],
        scratch_types=[
            pltpu.VMEM((CHUNK,), jnp.int32),
            pltpu.VMEM((CHUNK,), jnp.int32),
            pltpu.VMEM((CHUNK, D), F32),
            pltpu.VMEM((CHUNK, D), F32),
            pltpu.VMEM((CHUNK, B), F32),
            pltpu.SemaphoreType.DMA,
        ],
    )
    return f(xs2, xd, bht, row, col)


def _edge_body(gsa_ref, gsb_ref, ea_ref, bhe_ref,
               w1e_ref, w2_ref, b2_ref, w3_ref, b3_ref,
               er_ref, eo_ref, esum_ref, ecnt_ref):
    @pl.when(pl.program_id(0) == 0)
    def _():
        esum_ref[...] = jnp.zeros_like(esum_ref)
        ecnt_ref[...] = jnp.zeros_like(ecnt_ref)

    ea = ea_ref[...]
    h1 = jnp.maximum(gsa_ref[...] + gsb_ref[...] + _dot(ea, w1e_ref[...]), 0.0)
    h2 = jnp.maximum(_dot(h1, w2_ref[...]) + b2_ref[...], 0.0)
    er = _dot(h2, w3_ref[...]) + b3_ref[...]
    er_ref[...] = er
    eo_ref[...] = ea + er
    bhe = bhe_ref[...]
    esum_ref[...] += _dot_t(bhe, er)
    ecnt_ref[...] += _dot_t(bhe, jnp.ones_like(er))


@jax.jit
def _tc_edge(gsa, gsb, ea, bhe, w1e, w2, b2, w3, b3):
    nb = E // BLK_E
    return pl.pallas_call(
        _edge_body,
        grid=(nb,),
        in_specs=[
            pl.BlockSpec((BLK_E, D), lambda i: (i, 0)),
            pl.BlockSpec((BLK_E, D), lambda i: (i, 0)),
            pl.BlockSpec((BLK_E, D), lambda i: (i, 0)),
            pl.BlockSpec((BLK_E, B), lambda i: (i, 0)),
            pl.BlockSpec((D, D), lambda i: (0, 0)),
            pl.BlockSpec((D, D), lambda i: (0, 0)),
            pl.BlockSpec((1, D), lambda i: (0, 0)),
            pl.BlockSpec((D, D), lambda i: (0, 0)),
            pl.BlockSpec((1, D), lambda i: (0, 0)),
        ],
        out_specs=[
            pl.BlockSpec((BLK_E, D), lambda i: (i, 0)),
            pl.BlockSpec((BLK_E, D), lambda i: (i, 0)),
            pl.BlockSpec((B, D), lambda i: (0, 0)),
            pl.BlockSpec((B, D), lambda i: (0, 0)),
        ],
        out_shape=[
            jax.ShapeDtypeStruct((E, D), F32),
            jax.ShapeDtypeStruct((E, D), F32),
            jax.ShapeDtypeStruct((B, D), F32),
            jax.ShapeDtypeStruct((B, D), F32),
        ],
    )(gsa, gsb, ea, bhe, w1e, w2, b2, w3, b3)


def _sc_scatter_body(er, col, aggp,
                     i0, i1, i2, i3, i4, ebuf, acc):
    cid = lax.axis_index("c")
    sid = lax.axis_index("s")
    base = (cid * NS + sid) * TPE

    def zbody(r, _):
        for c in range(D // 16):
            ebuf[r, pl.ds(c * 16, 16)] = jnp.zeros((16,), F32)
        return 0
    lax.fori_loop(0, CHUNK, zbody, 0)
    pltpu.sync_copy(ebuf, acc.at[pl.ds(sid * RPT, CHUNK)])
    pltpu.sync_copy(ebuf.at[pl.ds(0, RPT - CHUNK)],
                    acc.at[pl.ds(sid * RPT + CHUNK, RPT - CHUNK)])
    plsc.subcore_barrier()

    idx = [i0, i1, i2, i3, i4]

    def body(i, _):
        off = pl.multiple_of(base + i * CHUNK, 8)
        pltpu.sync_copy(er.at[pl.ds(off, CHUNK)], ebuf)
        for k in range(NSUB):
            pltpu.sync_copy(col.at[pl.ds(off + k * SUB, SUB)], idx[k])
        for k in range(NSUB):
            pltpu.sync_copy(ebuf.at[pl.ds(k * SUB, SUB)],
                            acc.at[idx[k]], add=True)
        return 0

    lax.fori_loop(0, NIT, body, 0)
    plsc.subcore_barrier()
    pltpu.sync_copy(acc.at[pl.ds(sid * RPT, RPT)],
                    aggp.at[cid, pl.ds(sid * RPT, RPT)])


@jax.jit
def _sc_scatter(er, col):
    f = pl.kernel(
        _sc_scatter_body,
        mesh=plsc.VectorSubcoreMesh(core_axis_name="c", subcore_axis_name="s"),
        out_type=[jax.ShapeDtypeStruct((NC, N, D), F32)],
        scratch_types=[
            pltpu.VMEM((SUB,), jnp.int32),
            pltpu.VMEM((SUB,), jnp.int32),
            pltpu.VMEM((SUB,), jnp.int32),
            pltpu.VMEM((SUB,), jnp.int32),
            pltpu.VMEM((SUB,), jnp.int32),
            pltpu.VMEM((CHUNK, D), F32),
            pltpu.VMEM_SHARED((N, D), F32),
        ],
    )
    return f(er, col)


def _node_body(x_ref, a0_ref, a1_ref, bh_ref, u_ref,
               v1x_ref, v1a_ref, v1u_ref, c1_ref,
               v2_ref, c2_ref, v3_ref, c3_ref,
               xo_ref, nsum_ref, ncnt_ref):
    @pl.when(pl.program_id(0) == 0)
    def _():
        nsum_ref[...] = jnp.zeros_like(nsum_ref)
        ncnt_ref[...] = jnp.zeros_like(ncnt_ref)

    x = x_ref[...]
    agg = a0_ref[...] + a1_ref[...]
    bh = bh_ref[...]
    ub = _dot(u_ref[...], v1u_ref[...]) + c1_ref[...]
    h1 = jnp.maximum(_dot(x, v1x_ref[...]) + _dot(agg, v1a_ref[...])
                     + _dot(bh, ub), 0.0)
    h2 = jnp.maximum(_dot(h1, v2_ref[...]) + c2_ref[...], 0.0)
    xr = _dot(h2, v3_ref[...]) + c3_ref[...]
    xo_ref[...] = x + xr
    nsum_ref[...] += _dot_t(bh, xr)
    ncnt_ref[...] += _dot_t(bh, jnp.ones_like(xr))


@jax.jit
def _tc_node(x, aggf, bh, u, v1x, v1a, v1u, c1, v2, c2, v3, c3):
    nb = N // BLK_N
    return pl.pallas_call(
        _node_body,
        grid=(nb,),
        in_specs=[
            pl.BlockSpec((BLK_N, D), lambda i: (i, 0)),
            pl.BlockSpec((BLK_N, D), lambda i: (i, 0)),
            pl.BlockSpec((BLK_N, D), lambda i: (i + N // BLK_N, 0)),
            pl.BlockSpec((BLK_N, B), lambda i: (i, 0)),
            pl.BlockSpec((B, D), lambda i: (0, 0)),
            pl.BlockSpec((D, D), lambda i: (0, 0)),
            pl.BlockSpec((D, D), lambda i: (0, 0)),
            pl.BlockSpec((D, D), lambda i: (0, 0)),
            pl.BlockSpec((1, D), lambda i: (0, 0)),
            pl.BlockSpec((D, D), lambda i: (0, 0)),
            pl.BlockSpec((1, D), lambda i: (0, 0)),
            pl.BlockSpec((D, D), lambda i: (0, 0)),
            pl.BlockSpec((1, D), lambda i: (0, 0)),
        ],
        out_specs=[
            pl.BlockSpec((BLK_N, D), lambda i: (i, 0)),
            pl.BlockSpec((B, D), lambda i: (0, 0)),
            pl.BlockSpec((B, D), lambda i: (0, 0)),
        ],
        out_shape=[
            jax.ShapeDtypeStruct((N, D), F32),
            jax.ShapeDtypeStruct((B, D), F32),
            jax.ShapeDtypeStruct((B, D), F32),
        ],
    )(x, aggf, aggf, bh, u, v1x, v1a, v1u, c1, v2, c2, v3, c3)


def _glob_body(u_ref, nsum_ref, ncnt_ref, esum_ref, ecnt_ref,
               g1u_ref, g1n_ref, g1e_ref, g1b_ref,
               g2_ref, g2b_ref, g3_ref, g3b_ref, uo_ref):
    u = u_ref[...]
    nm = nsum_ref[...] / jnp.maximum(ncnt_ref[...], 1.0)
    em = esum_ref[...] / jnp.maximum(ecnt_ref[...], 1.0)
    h1 = jnp.maximum(_dot(u, g1u_ref[...]) + _dot(nm, g1n_ref[...])
                     + _dot(em, g1e_ref[...]) + g1b_ref[...], 0.0)
    h2 = jnp.maximum(_dot(h1, g2_ref[...]) + g2b_ref[...], 0.0)
    uo_ref[...] = u + _dot(h2, g3_ref[...]) + g3b_ref[...]


@jax.jit
def _tc_glob(u, nsum, ncnt, esum, ecnt, g1u, g1n, g1e, g1b, g2, g2b, g3, g3b):
    return pl.pallas_call(
        _glob_body,
        out_shape=jax.ShapeDtypeStruct((B, D), F32),
    )(u, nsum, ncnt, esum, ecnt, g1u, g1n, g1e, g1b, g2, g2b, g3, g3b)


def kernel(x, edge_index, edge_attr, u, batch, params):
    if u.ndim == 1:
        u = u[None]
    row = edge_index[0]
    col = edge_index[1]
    bh = (batch[:, None] == jnp.arange(B, dtype=batch.dtype)[None, :]
          ).astype(F32)

    for p in params:
        (w1, b1), (w2, b2), (w3, b3) = p['edge']
        (v1, c1), (v2, c2), (v3, c3) = p['node']
        (g1, g1b), (g2, g2b), (g3, g3b) = p['glob']
        w1s, w1d, w1e, w1u = w1[:D], w1[D:2 * D], w1[2 * D:3 * D], w1[3 * D:]
        v1x, v1a, v1u = v1[:D], v1[D:2 * D], v1[2 * D:]
        g1u, g1n, g1e = g1[:D], g1[D:2 * D], g1[2 * D:]

        xs2, xd = _tc_pre(x, bh, u, w1s, w1d, w1u, b1[None])
        gsa, gsb, bhe = _sc_gather(xs2, xd, bh, row, col)
        er, eo, esum, ecnt = _tc_edge(gsa, gsb, edge_attr, bhe,
                                      w1e, w2, b2[None], w3, b3[None])
        aggp = _sc_scatter(er, col)
        aggf = aggp.reshape(NC * N, D)
        xo, nsum, ncnt = _tc_node(x, aggf, bh, u, v1x, v1a, v1u, c1[None],
                                  v2, c2[None], v3, c3[None])
        uo = _tc_glob(u, nsum, ncnt, esum, ecnt, g1u, g1n, g1e, g1b[None],
                      g2, g2b[None], g3, g3b[None])
        x, edge_attr, u = xo, eo, uo

    return x, edge_attr, u


# speedup vs baseline: 4.5500x; 4.5500x over previous
import jax
import jax.numpy as jnp
from jax import lax
from jax.experimental import pallas as pl
from jax.experimental.pallas import tpu as pltpu
from jax.experimental.pallas import tpu_sc as plsc

F32 = jnp.float32

N = 10000
E = 320000
D = 128
B = 16

BLK_E = 2560
BLK_N = 2000
NBN = N // BLK_N

NC = 2
NS = 16
NW = NC * NS
CHUNK = 400
SUB = 80
NSUB = CHUNK // SUB
WB = 624
TAIL = N - NS * WB
CHUNK_S = 200
SUB_S = 40
NSUB_S = CHUNK_S // SUB_S


def _dot(a, b):
    return jnp.dot(a, b, preferred_element_type=F32)


def _dot_t(a, b):
    return lax.dot_general(a, b, (((0,), (0,)), ((), ())),
                           preferred_element_type=F32)


def _fill(ref, nrows, value):
    def zbody(r, _):
        for c in range(D // 16):
            ref[r, pl.ds(c * 16, 16)] = jnp.full((16,), value, F32)
        return 0
    lax.fori_loop(0, nrows, zbody, 0)


def _zero_acc_slice(zbuf, nb, acc, sid):
    off = 0
    while off < WB:
        span = min(nb, WB - off)
        pltpu.sync_copy(zbuf.at[pl.ds(0, span)],
                        acc.at[pl.ds(sid * WB + off, span)])
        off += span

    @pl.when(sid == NS - 1)
    def _():
        pltpu.sync_copy(zbuf.at[pl.ds(0, TAIL)], acc.at[pl.ds(NS * WB, TAIL)])


def _acc_writeback(acc, out, cid, sid):
    pltpu.sync_copy(acc.at[pl.ds(sid * WB, WB)],
                    out.at[cid, pl.ds(sid * WB, WB)])

    @pl.when(sid == NS - 1)
    def _():
        pltpu.sync_copy(acc.at[pl.ds(NS * WB, TAIL)],
                        out.at[cid, pl.ds(NS * WB, TAIL)])


def _pre_body(x_ref, bh_ref, u_ref, w1s_ref, w1d_ref, w1u_ref, b1_ref,
              xs2_ref, xd_ref):
    uu = _dot(u_ref[...], w1u_ref[...]) + b1_ref[...]
    xs2_ref[...] = _dot(x_ref[...], w1s_ref[...]) + _dot(bh_ref[...], uu)
    xd_ref[...] = _dot(x_ref[...], w1d_ref[...])


@jax.jit
def _tc_pre(x, bh, u, w1s, w1d, w1u, b1):
    return pl.pallas_call(
        _pre_body,
        grid=(NBN,),
        in_specs=[
            pl.BlockSpec((BLK_N, D), lambda i: (i, 0)),
            pl.BlockSpec((BLK_N, B), lambda i: (i, 0)),
            pl.BlockSpec((B, D), lambda i: (0, 0)),
            pl.BlockSpec((D, D), lambda i: (0, 0)),
            pl.BlockSpec((D, D), lambda i: (0, 0)),
            pl.BlockSpec((D, D), lambda i: (0, 0)),
            pl.BlockSpec((1, D), lambda i: (0, 0)),
        ],
        out_specs=[
            pl.BlockSpec((BLK_N, D), lambda i: (i, 0)),
            pl.BlockSpec((BLK_N, D), lambda i: (i, 0)),
        ],
        out_shape=[
            jax.ShapeDtypeStruct((N, D), F32),
            jax.ShapeDtypeStruct((N, D), F32),
        ],
    )(x, bh, u, w1s, w1d, w1u, b1)


def _sc_gather_body(xs2, xd, row, col,
                    gsa, gsb,
                    idxr, idxc, bufa, bufb, sem):
    wid = lax.axis_index("c") * NS + lax.axis_index("s")
    base = wid * (E // NW)

    def body(i, _):
        off = pl.multiple_of(base + i * CHUNK, 8)
        pltpu.sync_copy(row.at[pl.ds(off, CHUNK)], idxr)
        pltpu.sync_copy(col.at[pl.ds(off, CHUNK)], idxc)
        cps = []
        for k in range(NSUB):
            s = pl.ds(k * SUB, SUB)
            cps.append(pltpu.async_copy(xs2.at[idxr.at[s]], bufa.at[s], sem))
            cps.append(pltpu.async_copy(xd.at[idxc.at[s]], bufb.at[s], sem))
        for cp in cps:
            cp.wait()
        pltpu.sync_copy(bufa, gsa.at[pl.ds(off, CHUNK)])
        pltpu.sync_copy(bufb, gsb.at[pl.ds(off, CHUNK)])
        return 0

    lax.fori_loop(0, (E // NW) // CHUNK, body, 0)


@jax.jit
def _sc_gather(xs2, xd, row, col):
    f = pl.kernel(
        _sc_gather_body,
        mesh=plsc.VectorSubcoreMesh(core_axis_name="c", subcore_axis_name="s"),
        out_type=[
            jax.ShapeDtypeStruct((E, D), F32),
            jax.ShapeDtypeStruct((E, D), F32),
        ],
        scratch_types=[
            pltpu.VMEM((CHUNK,), jnp.int32),
            pltpu.VMEM((CHUNK,), jnp.int32),
            pltpu.VMEM((CHUNK, D), F32),
            pltpu.VMEM((CHUNK, D), F32),
            pltpu.SemaphoreType.DMA,
        ],
    )
    return f(xs2, xd, row, col)


def _edge_body(gsa_ref, gsb_ref, ea_ref,
               w1e_ref, w2_ref, b2_ref, w3_ref, b3_ref,
               er_ref, eo_ref):
    ea = ea_ref[...]
    h1 = jnp.maximum(gsa_ref[...] + gsb_ref[...] + _dot(ea, w1e_ref[...]), 0.0)
    h2 = jnp.maximum(_dot(h1, w2_ref[...]) + b2_ref[...], 0.0)
    er = _dot(h2, w3_ref[...]) + b3_ref[...]
    er_ref[...] = er
    eo_ref[...] = ea + er


@jax.jit
def _tc_edge(gsa, gsb, ea, w1e, w2, b2, w3, b3):
    return pl.pallas_call(
        _edge_body,
        grid=(E // BLK_E,),
        in_specs=[
            pl.BlockSpec((BLK_E, D), lambda i: (i, 0)),
            pl.BlockSpec((BLK_E, D), lambda i: (i, 0)),
            pl.BlockSpec((BLK_E, D), lambda i: (i, 0)),
            pl.BlockSpec((D, D), lambda i: (0, 0)),
            pl.BlockSpec((D, D), lambda i: (0, 0)),
            pl.BlockSpec((1, D), lambda i: (0, 0)),
            pl.BlockSpec((D, D), lambda i: (0, 0)),
            pl.BlockSpec((1, D), lambda i: (0, 0)),
        ],
        out_specs=[
            pl.BlockSpec((BLK_E, D), lambda i: (i, 0)),
            pl.BlockSpec((BLK_E, D), lambda i: (i, 0)),
        ],
        out_shape=[
            jax.ShapeDtypeStruct((E, D), F32),
            jax.ShapeDtypeStruct((E, D), F32),
        ],
    )(gsa, gsb, ea, w1e, w2, b2, w3, b3)


def _sc_scatter_body(er, rc, out,
                     i0, i1, i2, i3, i4, ebuf, acc):
    cid = lax.axis_index("c")
    sid = lax.axis_index("s")
    base = sid * (E // NS)

    _fill(ebuf, CHUNK_S, 0.0)
    _zero_acc_slice(ebuf, CHUNK_S, acc, sid)
    plsc.subcore_barrier()

    idx = [i0, i1, i2, i3, i4]

    def body(i, _):
        off = pl.multiple_of(base + i * CHUNK_S, 8)
        ioff = pl.multiple_of(cid * E + off, 8)
        pltpu.sync_copy(er.at[pl.ds(off, CHUNK_S)], ebuf)
        for k in range(NSUB_S):
            pltpu.sync_copy(rc.at[pl.ds(ioff + k * SUB_S, SUB_S)], idx[k])
        for k in range(NSUB_S):
            pltpu.sync_copy(ebuf.at[pl.ds(k * SUB_S, SUB_S)],
                            acc.at[idx[k]], add=True)
        return 0

    lax.fori_loop(0, (E // NS) // CHUNK_S, body, 0)
    plsc.subcore_barrier()
    _acc_writeback(acc, out, cid, sid)


@jax.jit
def _sc_scatter(er, rc):
    f = pl.kernel(
        _sc_scatter_body,
        mesh=plsc.VectorSubcoreMesh(core_axis_name="c", subcore_axis_name="s"),
        out_type=[jax.ShapeDtypeStruct((NC, N, D), F32)],
        scratch_types=[
            pltpu.VMEM((SUB_S,), jnp.int32),
            pltpu.VMEM((SUB_S,), jnp.int32),
            pltpu.VMEM((SUB_S,), jnp.int32),
            pltpu.VMEM((SUB_S,), jnp.int32),
            pltpu.VMEM((SUB_S,), jnp.int32),
            pltpu.VMEM((CHUNK_S, D), F32),
            pltpu.VMEM_SHARED((N, D), F32),
        ],
    )
    return f(er, rc)[0]


def _sc_hist_body(row, out,
                  i0, i1, i2, i3, i4, obuf, acc):
    cid = lax.axis_index("c")
    sid = lax.axis_index("s")
    base = (cid * NS + sid) * (E // NW)

    _fill(obuf, SUB, 0.0)
    _zero_acc_slice(obuf, SUB, acc, sid)
    plsc.subcore_barrier()
    _fill(obuf, SUB, 1.0)

    idx = [i0, i1, i2, i3, i4]

    def body(i, _):
        off = pl.multiple_of(base + i * CHUNK, 8)
        for k in range(NSUB):
            pltpu.sync_copy(row.at[pl.ds(off + k * SUB, SUB)], idx[k])
        for k in range(NSUB):
            pltpu.sync_copy(obuf.at[pl.ds(0, SUB)], acc.at[idx[k]], add=True)
        return 0

    lax.fori_loop(0, (E // NW) // CHUNK, body, 0)
    plsc.subcore_barrier()
    _acc_writeback(acc, out, cid, sid)


@jax.jit
def _sc_hist(row):
    f = pl.kernel(
        _sc_hist_body,
        mesh=plsc.VectorSubcoreMesh(core_axis_name="c", subcore_axis_name="s"),
        out_type=[jax.ShapeDtypeStruct((NC, N, D), F32)],
        scratch_types=[
            pltpu.VMEM((SUB,), jnp.int32),
            pltpu.VMEM((SUB,), jnp.int32),
            pltpu.VMEM((SUB,), jnp.int32),
            pltpu.VMEM((SUB,), jnp.int32),
            pltpu.VMEM((SUB,), jnp.int32),
            pltpu.VMEM((SUB, D), F32),
            pltpu.VMEM_SHARED((N, D), F32),
        ],
    )
    return f(row)[0]


def _counts_body(bh_ref, od0_ref, od1_ref, ncnt_ref, ecnt_ref):
    @pl.when(pl.program_id(0) == 0)
    def _():
        ncnt_ref[...] = jnp.zeros_like(ncnt_ref)
        ecnt_ref[...] = jnp.zeros_like(ecnt_ref)

    bh = bh_ref[...]
    ncnt_ref[...] += _dot_t(bh, jnp.ones((BLK_N, D), F32))
    ecnt_ref[...] += _dot_t(bh, od0_ref[...] + od1_ref[...])


@jax.jit
def _tc_counts(bh, odf):
    return pl.pallas_call(
        _counts_body,
        grid=(NBN,),
        in_specs=[
            pl.BlockSpec((BLK_N, B), lambda i: (i, 0)),
            pl.BlockSpec((BLK_N, D), lambda i: (i, 0)),
            pl.BlockSpec((BLK_N, D), lambda i: (i + NBN, 0)),
        ],
        out_specs=[
            pl.BlockSpec((B, D), lambda i: (0, 0)),
            pl.BlockSpec((B, D), lambda i: (0, 0)),
        ],
        out_shape=[
            jax.ShapeDtypeStruct((B, D), F32),
            jax.ShapeDtypeStruct((B, D), F32),
        ],
    )(bh, odf, odf)


def _node_body(x_ref, agg_ref, rs_ref, bh_ref, u_ref,
               v1x_ref, v1a_ref, v1u_ref, c1_ref,
               v2_ref, c2_ref, v3_ref, c3_ref,
               xo_ref, nsum_ref, esum_ref):
    @pl.when(pl.program_id(0) == 0)
    def _():
        nsum_ref[...] = jnp.zeros_like(nsum_ref)
        esum_ref[...] = jnp.zeros_like(esum_ref)

    x = x_ref[...]
    bh = bh_ref[...]
    ub = _dot(u_ref[...], v1u_ref[...]) + c1_ref[...]
    h1 = jnp.maximum(_dot(x, v1x_ref[...]) + _dot(agg_ref[...], v1a_ref[...])
                     + _dot(bh, ub), 0.0)
    h2 = jnp.maximum(_dot(h1, v2_ref[...]) + c2_ref[...], 0.0)
    xr = _dot(h2, v3_ref[...]) + c3_ref[...]
    xo_ref[...] = x + xr
    nsum_ref[...] += _dot_t(bh, xr)
    esum_ref[...] += _dot_t(bh, rs_ref[...])


@jax.jit
def _tc_node(x, aggf, bh, u, v1x, v1a, v1u, c1, v2, c2, v3, c3):
    return pl.pallas_call(
        _node_body,
        grid=(NBN,),
        in_specs=[
            pl.BlockSpec((BLK_N, D), lambda i: (i, 0)),
            pl.BlockSpec((BLK_N, D), lambda i: (i, 0)),
            pl.BlockSpec((BLK_N, D), lambda i: (i + NBN, 0)),
            pl.BlockSpec((BLK_N, B), lambda i: (i, 0)),
            pl.BlockSpec((B, D), lambda i: (0, 0)),
            pl.BlockSpec((D, D), lambda i: (0, 0)),
            pl.BlockSpec((D, D), lambda i: (0, 0)),
            pl.BlockSpec((D, D), lambda i: (0, 0)),
            pl.BlockSpec((1, D), lambda i: (0, 0)),
            pl.BlockSpec((D, D), lambda i: (0, 0)),
            pl.BlockSpec((1, D), lambda i: (0, 0)),
            pl.BlockSpec((D, D), lambda i: (0, 0)),
            pl.BlockSpec((1, D), lambda i: (0, 0)),
        ],
        out_specs=[
            pl.BlockSpec((BLK_N, D), lambda i: (i, 0)),
            pl.BlockSpec((B, D), lambda i: (0, 0)),
            pl.BlockSpec((B, D), lambda i: (0, 0)),
        ],
        out_shape=[
            jax.ShapeDtypeStruct((N, D), F32),
            jax.ShapeDtypeStruct((B, D), F32),
            jax.ShapeDtypeStruct((B, D), F32),
        ],
    )(x, aggf, aggf, bh, u, v1x, v1a, v1u, c1, v2, c2, v3, c3)


def _glob_body(u_ref, nsum_ref, ncnt_ref, esum_ref, ecnt_ref,
               g1u_ref, g1n_ref, g1e_ref, g1b_ref,
               g2_ref, g2b_ref, g3_ref, g3b_ref, uo_ref):
    u = u_ref[...]
    nm = nsum_ref[...] / jnp.maximum(ncnt_ref[...], 1.0)
    em = esum_ref[...] / jnp.maximum(ecnt_ref[...], 1.0)
    h1 = jnp.maximum(_dot(u, g1u_ref[...]) + _dot(nm, g1n_ref[...])
                     + _dot(em, g1e_ref[...]) + g1b_ref[...], 0.0)
    h2 = jnp.maximum(_dot(h1, g2_ref[...]) + g2b_ref[...], 0.0)
    uo_ref[...] = u + _dot(h2, g3_ref[...]) + g3b_ref[...]


@jax.jit
def _tc_glob(u, nsum, ncnt, esum, ecnt, g1u, g1n, g1e, g1b, g2, g2b, g3, g3b):
    return pl.pallas_call(
        _glob_body,
        out_shape=jax.ShapeDtypeStruct((B, D), F32),
    )(u, nsum, ncnt, esum, ecnt, g1u, g1n, g1e, g1b, g2, g2b, g3, g3b)


def kernel(x, edge_index, edge_attr, u, batch, params):
    if u.ndim == 1:
        u = u[None]
    row = edge_index[0]
    col = edge_index[1]
    rc = jnp.concatenate([col, row])
    bh = (batch[:, None] == jnp.arange(B, dtype=batch.dtype)[None, :]
          ).astype(F32)

    odeg = _sc_hist(row)
    ncnt, ecnt = _tc_counts(bh, odeg.reshape(NC * N, D))

    for p in params:
        (w1, b1), (w2, b2), (w3, b3) = p['edge']
        (v1, c1), (v2, c2), (v3, c3) = p['node']
        (g1, g1b), (g2, g2b), (g3, g3b) = p['glob']
        w1s, w1d, w1e, w1u = w1[:D], w1[D:2 * D], w1[2 * D:3 * D], w1[3 * D:]
        v1x, v1a, v1u = v1[:D], v1[D:2 * D], v1[2 * D:]
        g1u, g1n, g1e = g1[:D], g1[D:2 * D], g1[2 * D:]

        xs2, xd = _tc_pre(x, bh, u, w1s, w1d, w1u, b1[None])
        gsa, gsb = _sc_gather(xs2, xd, row, col)
        er, eo = _tc_edge(gsa, gsb, edge_attr, w1e, w2, b2[None], w3, b3[None])
        aggrs = _sc_scatter(er, rc)
        xo, nsum, esum = _tc_node(x, aggrs.reshape(NC * N, D), bh, u,
                                  v1x, v1a, v1u, c1[None],
                                  v2, c2[None], v3, c3[None])
        uo = _tc_glob(u, nsum, ncnt, esum, ecnt, g1u, g1n, g1e, g1b[None],
                      g2, g2b[None], g3, g3b[None])
        x, edge_attr, u = xo, eo, uo

    return x, edge_attr, u


# speedup vs baseline: 6.0911x; 1.3387x over previous
import jax
import jax.numpy as jnp
from jax import lax
from jax.experimental import pallas as pl
from jax.experimental.pallas import tpu as pltpu
from jax.experimental.pallas import tpu_sc as plsc

F32 = jnp.float32

N = 10000
E = 320000
D = 128
B = 16

BLK_E = 2560
BLK_N = 2000
NBN = N // BLK_N

NC = 2
NS = 16
NW = NC * NS
CHUNK = 400
SUB = 80
NSUB = CHUNK // SUB
WB = 624
TAIL = N - NS * WB
CHUNK_S = 160
SUB_S = 80
NSUB_S = CHUNK_S // SUB_S


def _dot(a, b):
    return jnp.dot(a, b, preferred_element_type=F32)


def _dot_t(a, b):
    return lax.dot_general(a, b, (((0,), (0,)), ((), ())),
                           preferred_element_type=F32)


def _fill(ref, nrows, value):
    def zbody(r, _):
        for c in range(D // 16):
            ref[r, pl.ds(c * 16, 16)] = jnp.full((16,), value, F32)
        return 0
    lax.fori_loop(0, nrows, zbody, 0)


def _zero_acc_slice(zbuf, nb, acc, sid):
    off = 0
    while off < WB:
        span = min(nb, WB - off)
        pltpu.sync_copy(zbuf.at[pl.ds(0, span)],
                        acc.at[pl.ds(sid * WB + off, span)])
        off += span

    @pl.when(sid == NS - 1)
    def _():
        pltpu.sync_copy(zbuf.at[pl.ds(0, TAIL)], acc.at[pl.ds(NS * WB, TAIL)])


def _acc_writeback(acc, out, cid, sid):
    pltpu.sync_copy(acc.at[pl.ds(sid * WB, WB)],
                    out.at[cid, pl.ds(sid * WB, WB)])

    @pl.when(sid == NS - 1)
    def _():
        pltpu.sync_copy(acc.at[pl.ds(NS * WB, TAIL)],
                        out.at[cid, pl.ds(NS * WB, TAIL)])


def _pre_body(x_ref, bh_ref, u_ref, w1s_ref, w1d_ref, w1u_ref, b1_ref,
              xs2_ref, xd_ref):
    uu = _dot(u_ref[...], w1u_ref[...]) + b1_ref[...]
    xs2_ref[...] = _dot(x_ref[...], w1s_ref[...]) + _dot(bh_ref[...], uu)
    xd_ref[...] = _dot(x_ref[...], w1d_ref[...])


@jax.jit
def _tc_pre(x, bh, u, w1s, w1d, w1u, b1):
    return pl.pallas_call(
        _pre_body,
        grid=(NBN,),
        in_specs=[
            pl.BlockSpec((BLK_N, D), lambda i: (i, 0)),
            pl.BlockSpec((BLK_N, B), lambda i: (i, 0)),
            pl.BlockSpec((B, D), lambda i: (0, 0)),
            pl.BlockSpec((D, D), lambda i: (0, 0)),
            pl.BlockSpec((D, D), lambda i: (0, 0)),
            pl.BlockSpec((D, D), lambda i: (0, 0)),
            pl.BlockSpec((1, D), lambda i: (0, 0)),
        ],
        out_specs=[
            pl.BlockSpec((BLK_N, D), lambda i: (i, 0)),
            pl.BlockSpec((BLK_N, D), lambda i: (i, 0)),
        ],
        out_shape=[
            jax.ShapeDtypeStruct((N, D), F32),
            jax.ShapeDtypeStruct((N, D), F32),
        ],
    )(x, bh, u, w1s, w1d, w1u, b1)


GCH = 80
GNB = 5
GROUNDS = (E // NW) // GCH // GNB


def _sc_gather_body(xs2, xd, row, col, gsa, gsb, *scr):
    idxr = scr[0:GNB]
    idxc = scr[GNB:2 * GNB]
    bufa = scr[2 * GNB:3 * GNB]
    bufb = scr[3 * GNB:4 * GNB]
    gsem = scr[4 * GNB:5 * GNB]
    wsem = scr[5 * GNB:6 * GNB]
    isem = scr[6 * GNB:7 * GNB]
    wid = lax.axis_index("c") * NS + lax.axis_index("s")
    base = wid * (E // NW)

    def off_of(i):
        return pl.multiple_of(base + i * GCH, 8)

    def fire_idx(b, i):
        off = off_of(i)
        pltpu.async_copy(row.at[pl.ds(off, GCH)], idxr[b], isem[b])
        pltpu.async_copy(col.at[pl.ds(off, GCH)], idxc[b], isem[b])

    def wait_idx(b):
        pltpu.make_async_copy(row.at[pl.ds(base, GCH)], idxr[b],
                              isem[b]).wait()
        pltpu.make_async_copy(col.at[pl.ds(base, GCH)], idxc[b],
                              isem[b]).wait()

    def fire_gather(b):
        pltpu.async_copy(xs2.at[idxr[b]], bufa[b], gsem[b])
        pltpu.async_copy(xd.at[idxc[b]], bufb[b], gsem[b])

    def wait_gather(b):
        pltpu.make_async_copy(xs2.at[idxr[b]], bufa[b], gsem[b]).wait()
        pltpu.make_async_copy(xd.at[idxc[b]], bufb[b], gsem[b]).wait()

    def fire_write(b, i):
        off = off_of(i)
        pltpu.async_copy(bufa[b], gsa.at[pl.ds(off, GCH)], wsem[b])
        pltpu.async_copy(bufb[b], gsb.at[pl.ds(off, GCH)], wsem[b])

    def wait_write(b):
        pltpu.make_async_copy(bufa[b], gsa.at[pl.ds(base, GCH)],
                              wsem[b]).wait()
        pltpu.make_async_copy(bufb[b], gsb.at[pl.ds(base, GCH)],
                              wsem[b]).wait()

    for b in range(GNB):
        fire_idx(b, b)
    for b in range(GNB):
        wait_idx(b)
        fire_gather(b)

    def round_body(j, _):
        not_last = j < GROUNDS - 1
        for b in range(GNB):
            i = j * GNB + b
            wait_gather(b)
            fire_write(b, i)

            @pl.when(not_last)
            def _(b=b, i=i):
                fire_idx(b, i + GNB)
        for b in range(GNB):
            @pl.when(not_last)
            def _(b=b):
                wait_write(b)
                wait_idx(b)
                fire_gather(b)
        return 0

    lax.fori_loop(0, GROUNDS, round_body, 0)
    for b in range(GNB):
        wait_write(b)


@jax.jit
def _sc_gather(xs2, xd, row, col):
    scratch = (
        [pltpu.VMEM((GCH,), jnp.int32) for _ in range(2 * GNB)]
        + [pltpu.VMEM((GCH, D), F32) for _ in range(2 * GNB)]
        + [pltpu.SemaphoreType.DMA for _ in range(3 * GNB)]
    )
    f = pl.kernel(
        _sc_gather_body,
        mesh=plsc.VectorSubcoreMesh(core_axis_name="c", subcore_axis_name="s"),
        out_type=[
            jax.ShapeDtypeStruct((E, D), F32),
            jax.ShapeDtypeStruct((E, D), F32),
        ],
        scratch_types=scratch,
    )
    return f(xs2, xd, row, col)


def _edge_body(gsa_ref, gsb_ref, ea_ref,
               w1e_ref, w2_ref, b2_ref, w3_ref, b3_ref,
               er_ref, eo_ref):
    ea = ea_ref[...]
    h1 = jnp.maximum(gsa_ref[...] + gsb_ref[...] + _dot(ea, w1e_ref[...]), 0.0)
    h2 = jnp.maximum(_dot(h1, w2_ref[...]) + b2_ref[...], 0.0)
    er = _dot(h2, w3_ref[...]) + b3_ref[...]
    er_ref[...] = er
    eo_ref[...] = ea + er


@jax.jit
def _tc_edge(gsa, gsb, ea, w1e, w2, b2, w3, b3):
    return pl.pallas_call(
        _edge_body,
        grid=(E // BLK_E,),
        in_specs=[
            pl.BlockSpec((BLK_E, D), lambda i: (i, 0)),
            pl.BlockSpec((BLK_E, D), lambda i: (i, 0)),
            pl.BlockSpec((BLK_E, D), lambda i: (i, 0)),
            pl.BlockSpec((D, D), lambda i: (0, 0)),
            pl.BlockSpec((D, D), lambda i: (0, 0)),
            pl.BlockSpec((1, D), lambda i: (0, 0)),
            pl.BlockSpec((D, D), lambda i: (0, 0)),
            pl.BlockSpec((1, D), lambda i: (0, 0)),
        ],
        out_specs=[
            pl.BlockSpec((BLK_E, D), lambda i: (i, 0)),
            pl.BlockSpec((BLK_E, D), lambda i: (i, 0)),
        ],
        out_shape=[
            jax.ShapeDtypeStruct((E, D), F32),
            jax.ShapeDtypeStruct((E, D), F32),
        ],
    )(gsa, gsb, ea, w1e, w2, b2, w3, b3)


def _sc_scatter_body(er, rc, out, *scr):
    idx = [scr[0:NSUB_S], scr[NSUB_S:2 * NSUB_S]]
    ebuf = list(scr[2 * NSUB_S:2 * NSUB_S + 2])
    esem = scr[2 * NSUB_S + 2:2 * NSUB_S + 4]
    isem = scr[2 * NSUB_S + 4:2 * NSUB_S + 6]
    ssem = scr[2 * NSUB_S + 6:2 * NSUB_S + 8]
    acc = scr[-1]
    cid = lax.axis_index("c")
    sid = lax.axis_index("s")
    base = sid * (E // NS)
    nit = (E // NS) // CHUNK_S
    nring = nit - 1

    _fill(ebuf[0], CHUNK_S, 0.0)
    _zero_acc_slice(ebuf[0], CHUNK_S, acc, sid)
    plsc.subcore_barrier()

    def off_of(i):
        return pl.multiple_of(base + i * CHUNK_S, 8)

    def fire_er(b, i):
        pltpu.async_copy(er.at[pl.ds(off_of(i), CHUNK_S)], ebuf[b], esem[b])

    def wait_er(b):
        pltpu.make_async_copy(er.at[pl.ds(base, CHUNK_S)], ebuf[b],
                              esem[b]).wait()

    def fire_idx(b, i):
        ioff = pl.multiple_of(cid * E + off_of(i), 8)
        for k in range(NSUB_S):
            pltpu.async_copy(rc.at[pl.ds(ioff + k * SUB_S, SUB_S)],
                             idx[b][k], isem[b])

    def wait_idx(b):
        for k in range(NSUB_S):
            pltpu.make_async_copy(rc.at[pl.ds(base, SUB_S)], idx[b][k],
                                  isem[b]).wait()

    def fire_sc(b):
        for k in range(NSUB_S):
            pltpu.async_copy(ebuf[b].at[pl.ds(k * SUB_S, SUB_S)],
                             acc.at[idx[b][k]], ssem[b], add=True)

    def wait_sc(b):
        for k in range(NSUB_S):
            pltpu.make_async_copy(ebuf[b].at[pl.ds(k * SUB_S, SUB_S)],
                                  acc.at[idx[b][k]], ssem[b]).wait()

    for b in range(2):
        fire_er(b, b)
        fire_idx(b, b)

    def round_body(j, _):
        for b in range(2):
            wait_er(b)
            wait_idx(b)
            fire_sc(b)
        for b in range(2):
            i_next = j * 2 + b + 2

            @pl.when(i_next < nring)
            def _(b=b, i_next=i_next):
                wait_sc(b)
                fire_er(b, i_next)
                fire_idx(b, i_next)
        return 0

    lax.fori_loop(0, nring // 2, round_body, 0)
    for b in range(2):
        wait_sc(b)

    i = nit - 1
    pltpu.sync_copy(er.at[pl.ds(off_of(i), CHUNK_S)], ebuf[0])
    ioff = pl.multiple_of(cid * E + off_of(i), 8)
    for k in range(NSUB_S):
        pltpu.sync_copy(rc.at[pl.ds(ioff + k * SUB_S, SUB_S)], idx[0][k])
    for k in range(NSUB_S):
        pltpu.sync_copy(ebuf[0].at[pl.ds(k * SUB_S, SUB_S)],
                        acc.at[idx[0][k]], add=True)

    plsc.subcore_barrier()
    _acc_writeback(acc, out, cid, sid)


@jax.jit
def _sc_scatter(er, rc):
    scratch = (
        [pltpu.VMEM((SUB_S,), jnp.int32) for _ in range(2 * NSUB_S)]
        + [pltpu.VMEM((CHUNK_S, D), F32) for _ in range(2)]
        + [pltpu.SemaphoreType.DMA for _ in range(6)]
        + [pltpu.VMEM_SHARED((N, D), F32)]
    )
    f = pl.kernel(
        _sc_scatter_body,
        mesh=plsc.VectorSubcoreMesh(core_axis_name="c", subcore_axis_name="s"),
        out_type=[jax.ShapeDtypeStruct((NC, N, D), F32)],
        scratch_types=scratch,
    )
    return f(er, rc)[0]


def _sc_hist_body(row, out,
                  i0, i1, i2, i3, i4, obuf, acc):
    cid = lax.axis_index("c")
    sid = lax.axis_index("s")
    base = (cid * NS + sid) * (E // NW)

    _fill(obuf, SUB, 0.0)
    _zero_acc_slice(obuf, SUB, acc, sid)
    plsc.subcore_barrier()
    _fill(obuf, SUB, 1.0)

    idx = [i0, i1, i2, i3, i4]

    def body(i, _):
        off = pl.multiple_of(base + i * CHUNK, 8)
        for k in range(NSUB):
            pltpu.sync_copy(row.at[pl.ds(off + k * SUB, SUB)], idx[k])
        for k in range(NSUB):
            pltpu.sync_copy(obuf.at[pl.ds(0, SUB)], acc.at[idx[k]], add=True)
        return 0

    lax.fori_loop(0, (E // NW) // CHUNK, body, 0)
    plsc.subcore_barrier()
    _acc_writeback(acc, out, cid, sid)


@jax.jit
def _sc_hist(row):
    f = pl.kernel(
        _sc_hist_body,
        mesh=plsc.VectorSubcoreMesh(core_axis_name="c", subcore_axis_name="s"),
        out_type=[jax.ShapeDtypeStruct((NC, N, D), F32)],
        scratch_types=[
            pltpu.VMEM((SUB,), jnp.int32),
            pltpu.VMEM((SUB,), jnp.int32),
            pltpu.VMEM((SUB,), jnp.int32),
            pltpu.VMEM((SUB,), jnp.int32),
            pltpu.VMEM((SUB,), jnp.int32),
            pltpu.VMEM((SUB, D), F32),
            pltpu.VMEM_SHARED((N, D), F32),
        ],
    )
    return f(row)[0]


def _counts_body(bh_ref, od0_ref, od1_ref, ncnt_ref, ecnt_ref):
    @pl.when(pl.program_id(0) == 0)
    def _():
        ncnt_ref[...] = jnp.zeros_like(ncnt_ref)
        ecnt_ref[...] = jnp.zeros_like(ecnt_ref)

    bh = bh_ref[...]
    ncnt_ref[...] += _dot_t(bh, jnp.ones((BLK_N, D), F32))
    ecnt_ref[...] += _dot_t(bh, od0_ref[...] + od1_ref[...])


@jax.jit
def _tc_counts(bh, odf):
    return pl.pallas_call(
        _counts_body,
        grid=(NBN,),
        in_specs=[
            pl.BlockSpec((BLK_N, B), lambda i: (i, 0)),
            pl.BlockSpec((BLK_N, D), lambda i: (i, 0)),
            pl.BlockSpec((BLK_N, D), lambda i: (i + NBN, 0)),
        ],
        out_specs=[
            pl.BlockSpec((B, D), lambda i: (0, 0)),
            pl.BlockSpec((B, D), lambda i: (0, 0)),
        ],
        out_shape=[
            jax.ShapeDtypeStruct((B, D), F32),
            jax.ShapeDtypeStruct((B, D), F32),
        ],
    )(bh, odf, odf)


def _node_body(x_ref, agg_ref, rs_ref, bh_ref, u_ref,
               v1x_ref, v1a_ref, v1u_ref, c1_ref,
               v2_ref, c2_ref, v3_ref, c3_ref,
               xo_ref, nsum_ref, esum_ref):
    @pl.when(pl.program_id(0) == 0)
    def _():
        nsum_ref[...] = jnp.zeros_like(nsum_ref)
        esum_ref[...] = jnp.zeros_like(esum_ref)

    x = x_ref[...]
    bh = bh_ref[...]
    ub = _dot(u_ref[...], v1u_ref[...]) + c1_ref[...]
    h1 = jnp.maximum(_dot(x, v1x_ref[...]) + _dot(agg_ref[...], v1a_ref[...])
                     + _dot(bh, ub), 0.0)
    h2 = jnp.maximum(_dot(h1, v2_ref[...]) + c2_ref[...], 0.0)
    xr = _dot(h2, v3_ref[...]) + c3_ref[...]
    xo_ref[...] = x + xr
    nsum_ref[...] += _dot_t(bh, xr)
    esum_ref[...] += _dot_t(bh, rs_ref[...])


@jax.jit
def _tc_node(x, aggf, bh, u, v1x, v1a, v1u, c1, v2, c2, v3, c3):
    return pl.pallas_call(
        _node_body,
        grid=(NBN,),
        in_specs=[
            pl.BlockSpec((BLK_N, D), lambda i: (i, 0)),
            pl.BlockSpec((BLK_N, D), lambda i: (i, 0)),
            pl.BlockSpec((BLK_N, D), lambda i: (i + NBN, 0)),
            pl.BlockSpec((BLK_N, B), lambda i: (i, 0)),
            pl.BlockSpec((B, D), lambda i: (0, 0)),
            pl.BlockSpec((D, D), lambda i: (0, 0)),
            pl.BlockSpec((D, D), lambda i: (0, 0)),
            pl.BlockSpec((D, D), lambda i: (0, 0)),
            pl.BlockSpec((1, D), lambda i: (0, 0)),
            pl.BlockSpec((D, D), lambda i: (0, 0)),
            pl.BlockSpec((1, D), lambda i: (0, 0)),
            pl.BlockSpec((D, D), lambda i: (0, 0)),
            pl.BlockSpec((1, D), lambda i: (0, 0)),
        ],
        out_specs=[
            pl.BlockSpec((BLK_N, D), lambda i: (i, 0)),
            pl.BlockSpec((B, D), lambda i: (0, 0)),
            pl.BlockSpec((B, D), lambda i: (0, 0)),
        ],
        out_shape=[
            jax.ShapeDtypeStruct((N, D), F32),
            jax.ShapeDtypeStruct((B, D), F32),
            jax.ShapeDtypeStruct((B, D), F32),
        ],
    )(x, aggf, aggf, bh, u, v1x, v1a, v1u, c1, v2, c2, v3, c3)


def _glob_body(u_ref, nsum_ref, ncnt_ref, esum_ref, ecnt_ref,
               g1u_ref, g1n_ref, g1e_ref, g1b_ref,
               g2_ref, g2b_ref, g3_ref, g3b_ref, uo_ref):
    u = u_ref[...]
    nm = nsum_ref[...] / jnp.maximum(ncnt_ref[...], 1.0)
    em = esum_ref[...] / jnp.maximum(ecnt_ref[...], 1.0)
    h1 = jnp.maximum(_dot(u, g1u_ref[...]) + _dot(nm, g1n_ref[...])
                     + _dot(em, g1e_ref[...]) + g1b_ref[...], 0.0)
    h2 = jnp.maximum(_dot(h1, g2_ref[...]) + g2b_ref[...], 0.0)
    uo_ref[...] = u + _dot(h2, g3_ref[...]) + g3b_ref[...]


@jax.jit
def _tc_glob(u, nsum, ncnt, esum, ecnt, g1u, g1n, g1e, g1b, g2, g2b, g3, g3b):
    return pl.pallas_call(
        _glob_body,
        out_shape=jax.ShapeDtypeStruct((B, D), F32),
    )(u, nsum, ncnt, esum, ecnt, g1u, g1n, g1e, g1b, g2, g2b, g3, g3b)


def kernel(x, edge_index, edge_attr, u, batch, params):
    if u.ndim == 1:
        u = u[None]
    row = edge_index[0]
    col = edge_index[1]
    rc = jnp.concatenate([col, row])
    bh = (batch[:, None] == jnp.arange(B, dtype=batch.dtype)[None, :]
          ).astype(F32)

    odeg = _sc_hist(row)
    ncnt, ecnt = _tc_counts(bh, odeg.reshape(NC * N, D))

    for p in params:
        (w1, b1), (w2, b2), (w3, b3) = p['edge']
        (v1, c1), (v2, c2), (v3, c3) = p['node']
        (g1, g1b), (g2, g2b), (g3, g3b) = p['glob']
        w1s, w1d, w1e, w1u = w1[:D], w1[D:2 * D], w1[2 * D:3 * D], w1[3 * D:]
        v1x, v1a, v1u = v1[:D], v1[D:2 * D], v1[2 * D:]
        g1u, g1n, g1e = g1[:D], g1[D:2 * D], g1[2 * D:]

        xs2, xd = _tc_pre(x, bh, u, w1s, w1d, w1u, b1[None])
        gsa, gsb = _sc_gather(xs2, xd, row, col)
        er, eo = _tc_edge(gsa, gsb, edge_attr, w1e, w2, b2[None], w3, b3[None])
        aggrs = _sc_scatter(er, rc)
        xo, nsum, esum = _tc_node(x, aggrs.reshape(NC * N, D), bh, u,
                                  v1x, v1a, v1u, c1[None],
                                  v2, c2[None], v3, c3[None])
        uo = _tc_glob(u, nsum, ncnt, esum, ecnt, g1u, g1n, g1e, g1b[None],
                      g2, g2b[None], g3, g3b[None])
        x, edge_attr, u = xo, eo, uo

    return x, edge_attr, u


# speedup vs baseline: 7.4873x; 1.2292x over previous
import jax
import jax.numpy as jnp
from jax import lax
from jax.experimental import pallas as pl
from jax.experimental.pallas import tpu as pltpu
from jax.experimental.pallas import tpu_sc as plsc

F32 = jnp.float32

N = 10000
E = 320000
D = 128
B = 16

BLK_E = 2560
BLK_N = 2000
NBN = N // BLK_N

NC = 2
NS = 16
NW = NC * NS
CHUNK = 400
SUB = 80
NSUB = CHUNK // SUB
WB = 624
TAIL = N - NS * WB
CHUNK_S = 80
NB_S = 4


def _dot(a, b):
    return jnp.dot(a, b, preferred_element_type=F32)


def _dot_t(a, b):
    return lax.dot_general(a, b, (((0,), (0,)), ((), ())),
                           preferred_element_type=F32)


def _fill(ref, nrows, value):
    def zbody(r, _):
        for c in range(D // 16):
            ref[r, pl.ds(c * 16, 16)] = jnp.full((16,), value, F32)
        return 0
    lax.fori_loop(0, nrows, zbody, 0)


def _zero_acc_slice(zbuf, nb, acc, sid):
    off = 0
    while off < WB:
        span = min(nb, WB - off)
        pltpu.sync_copy(zbuf.at[pl.ds(0, span)],
                        acc.at[pl.ds(sid * WB + off, span)])
        off += span

    @pl.when(sid == NS - 1)
    def _():
        pltpu.sync_copy(zbuf.at[pl.ds(0, TAIL)], acc.at[pl.ds(NS * WB, TAIL)])


def _acc_writeback(acc, out, cid, sid):
    pltpu.sync_copy(acc.at[pl.ds(sid * WB, WB)],
                    out.at[cid, pl.ds(sid * WB, WB)])

    @pl.when(sid == NS - 1)
    def _():
        pltpu.sync_copy(acc.at[pl.ds(NS * WB, TAIL)],
                        out.at[cid, pl.ds(NS * WB, TAIL)])


def _pre_body(x_ref, bh_ref, u_ref, w1s_ref, w1d_ref, w1u_ref, b1_ref,
              xs2_ref, xd_ref):
    uu = _dot(u_ref[...], w1u_ref[...]) + b1_ref[...]
    xs2_ref[...] = _dot(x_ref[...], w1s_ref[...]) + _dot(bh_ref[...], uu)
    xd_ref[...] = _dot(x_ref[...], w1d_ref[...])


@jax.jit
def _tc_pre(x, bh, u, w1s, w1d, w1u, b1):
    return pl.pallas_call(
        _pre_body,
        grid=(NBN,),
        in_specs=[
            pl.BlockSpec((BLK_N, D), lambda i: (i, 0)),
            pl.BlockSpec((BLK_N, B), lambda i: (i, 0)),
            pl.BlockSpec((B, D), lambda i: (0, 0)),
            pl.BlockSpec((D, D), lambda i: (0, 0)),
            pl.BlockSpec((D, D), lambda i: (0, 0)),
            pl.BlockSpec((D, D), lambda i: (0, 0)),
            pl.BlockSpec((1, D), lambda i: (0, 0)),
        ],
        out_specs=[
            pl.BlockSpec((BLK_N, D), lambda i: (i, 0)),
            pl.BlockSpec((BLK_N, D), lambda i: (i, 0)),
        ],
        out_shape=[
            jax.ShapeDtypeStruct((N, D), F32),
            jax.ShapeDtypeStruct((N, D), F32),
        ],
    )(x, bh, u, w1s, w1d, w1u, b1)


GCH = 80
GNB = 5
GROUNDS = (E // NW) // GCH // GNB


def _sc_gather_body(xs2, xd, row, col, gs, *scr):
    idxr = scr[0:GNB]
    idxc = scr[GNB:2 * GNB]
    buf = scr[2 * GNB:3 * GNB]
    gsem = scr[3 * GNB:4 * GNB]
    asem = scr[4 * GNB:5 * GNB]
    wsem = scr[5 * GNB:6 * GNB]
    isem = scr[6 * GNB:7 * GNB]
    wid = lax.axis_index("c") * NS + lax.axis_index("s")
    base = wid * (E // NW)

    def off_of(i):
        return pl.multiple_of(base + i * GCH, 8)

    def fire_idx(b, i):
        off = off_of(i)
        pltpu.async_copy(row.at[pl.ds(off, GCH)], idxr[b], isem[b])
        pltpu.async_copy(col.at[pl.ds(off, GCH)], idxc[b], isem[b])

    def wait_idx(b):
        pltpu.make_async_copy(row.at[pl.ds(base, GCH)], idxr[b],
                              isem[b]).wait()
        pltpu.make_async_copy(col.at[pl.ds(base, GCH)], idxc[b],
                              isem[b]).wait()

    def fire_g1(b):
        pltpu.async_copy(xs2.at[idxr[b]], buf[b], gsem[b])

    def wait_g1(b):
        pltpu.make_async_copy(xs2.at[idxr[b]], buf[b], gsem[b]).wait()

    def fire_g2(b):
        pltpu.async_copy(xd.at[idxc[b]], buf[b], asem[b], add=True)

    def wait_g2(b):
        pltpu.make_async_copy(xd.at[idxc[b]], buf[b], asem[b]).wait()

    def fire_write(b, i):
        pltpu.async_copy(buf[b], gs.at[pl.ds(off_of(i), GCH)], wsem[b])

    def wait_write(b):
        pltpu.make_async_copy(buf[b], gs.at[pl.ds(base, GCH)],
                              wsem[b]).wait()

    for b in range(GNB):
        fire_idx(b, b)
    for b in range(GNB):
        wait_idx(b)
        fire_g1(b)
    for b in range(GNB):
        wait_g1(b)
        fire_g2(b)

    def round_body(j, _):
        not_last = j < GROUNDS - 1
        for b in range(GNB):
            i = j * GNB + b
            wait_g2(b)
            fire_write(b, i)

            @pl.when(not_last)
            def _(b=b, i=i):
                fire_idx(b, i + GNB)
        for b in range(GNB):
            @pl.when(not_last)
            def _(b=b):
                wait_write(b)
                wait_idx(b)
                fire_g1(b)
        for b in range(GNB):
            @pl.when(not_last)
            def _(b=b):
                wait_g1(b)
                fire_g2(b)
        return 0

    lax.fori_loop(0, GROUNDS, round_body, 0)
    for b in range(GNB):
        wait_write(b)


@jax.jit
def _sc_gather(xs2, xd, row, col):
    scratch = (
        [pltpu.VMEM((GCH,), jnp.int32) for _ in range(2 * GNB)]
        + [pltpu.VMEM((GCH, D), F32) for _ in range(GNB)]
        + [pltpu.SemaphoreType.DMA for _ in range(4 * GNB)]
    )
    f = pl.kernel(
        _sc_gather_body,
        mesh=plsc.VectorSubcoreMesh(core_axis_name="c", subcore_axis_name="s"),
        out_type=[jax.ShapeDtypeStruct((E, D), F32)],
        scratch_types=scratch,
    )
    return f(xs2, xd, row, col)[0]


def _edge_body(gs_ref, ea_ref,
               w1e_ref, w2_ref, b2_ref, w3_ref, b3_ref,
               er_ref, eo_ref):
    ea = ea_ref[...]
    h1 = jnp.maximum(gs_ref[...] + _dot(ea, w1e_ref[...]), 0.0)
    h2 = jnp.maximum(_dot(h1, w2_ref[...]) + b2_ref[...], 0.0)
    er = _dot(h2, w3_ref[...]) + b3_ref[...]
    er_ref[...] = er
    eo_ref[...] = ea + er


@jax.jit
def _tc_edge(gs, ea, w1e, w2, b2, w3, b3):
    return pl.pallas_call(
        _edge_body,
        grid=(E // BLK_E,),
        in_specs=[
            pl.BlockSpec((BLK_E, D), lambda i: (i, 0)),
            pl.BlockSpec((BLK_E, D), lambda i: (i, 0)),
            pl.BlockSpec((D, D), lambda i: (0, 0)),
            pl.BlockSpec((D, D), lambda i: (0, 0)),
            pl.BlockSpec((1, D), lambda i: (0, 0)),
            pl.BlockSpec((D, D), lambda i: (0, 0)),
            pl.BlockSpec((1, D), lambda i: (0, 0)),
        ],
        out_specs=[
            pl.BlockSpec((BLK_E, D), lambda i: (i, 0)),
            pl.BlockSpec((BLK_E, D), lambda i: (i, 0)),
        ],
        out_shape=[
            jax.ShapeDtypeStruct((E, D), F32),
            jax.ShapeDtypeStruct((E, D), F32),
        ],
    )(gs, ea, w1e, w2, b2, w3, b3)


def _sc_scatter_body(er, rc, out, *scr):
    idx = scr[0:NB_S]
    ebuf = scr[NB_S:2 * NB_S]
    esem = scr[2 * NB_S:3 * NB_S]
    isem = scr[3 * NB_S:4 * NB_S]
    ssem = scr[4 * NB_S:5 * NB_S]
    acc = scr[-1]
    cid = lax.axis_index("c")
    sid = lax.axis_index("s")
    base = sid * (E // NS)
    nit = (E // NS) // CHUNK_S
    nring = nit - 2

    _fill(ebuf[0], CHUNK_S, 0.0)
    _zero_acc_slice(ebuf[0], CHUNK_S, acc, sid)
    plsc.subcore_barrier()

    def off_of(i):
        return pl.multiple_of(base + i * CHUNK_S, 8)

    def fire_er(b, i):
        pltpu.async_copy(er.at[pl.ds(off_of(i), CHUNK_S)], ebuf[b], esem[b])

    def wait_er(b):
        pltpu.make_async_copy(er.at[pl.ds(base, CHUNK_S)], ebuf[b],
                              esem[b]).wait()

    def fire_idx(b, i):
        ioff = pl.multiple_of(cid * E + off_of(i), 8)
        pltpu.async_copy(rc.at[pl.ds(ioff, CHUNK_S)], idx[b], isem[b])

    def wait_idx(b):
        pltpu.make_async_copy(rc.at[pl.ds(base, CHUNK_S)], idx[b],
                              isem[b]).wait()

    def fire_sc(b):
        pltpu.async_copy(ebuf[b], acc.at[idx[b]], ssem[b], add=True)

    def wait_sc(b):
        pltpu.make_async_copy(ebuf[b], acc.at[idx[b]], ssem[b]).wait()

    for b in range(NB_S):
        fire_er(b, b)
        fire_idx(b, b)

    def round_body(j, _):
        for b in range(NB_S):
            wait_er(b)
            wait_idx(b)
            fire_sc(b)
        for b in range(NB_S):
            i_next = j * NB_S + b + NB_S

            @pl.when(i_next < nring)
            def _(b=b, i_next=i_next):
                wait_sc(b)
                fire_er(b, i_next)
                fire_idx(b, i_next)
        return 0

    lax.fori_loop(0, nring // NB_S, round_body, 0)
    for b in range(NB_S):
        wait_sc(b)

    for i in range(nit - 2, nit):
        pltpu.sync_copy(er.at[pl.ds(off_of(i), CHUNK_S)], ebuf[0])
        ioff = pl.multiple_of(cid * E + off_of(i), 8)
        pltpu.sync_copy(rc.at[pl.ds(ioff, CHUNK_S)], idx[0])
        pltpu.sync_copy(ebuf[0], acc.at[idx[0]], add=True)

    plsc.subcore_barrier()
    _acc_writeback(acc, out, cid, sid)


@jax.jit
def _sc_scatter(er, rc):
    scratch = (
        [pltpu.VMEM((CHUNK_S,), jnp.int32) for _ in range(NB_S)]
        + [pltpu.VMEM((CHUNK_S, D), F32) for _ in range(NB_S)]
        + [pltpu.SemaphoreType.DMA for _ in range(3 * NB_S)]
        + [pltpu.VMEM_SHARED((N, D), F32)]
    )
    f = pl.kernel(
        _sc_scatter_body,
        mesh=plsc.VectorSubcoreMesh(core_axis_name="c", subcore_axis_name="s"),
        out_type=[jax.ShapeDtypeStruct((NC, N, D), F32)],
        scratch_types=scratch,
    )
    return f(er, rc)[0]


NB_H = 4


def _sc_hist_body(row, out, *scr):
    idx = scr[0:NB_H]
    obuf = scr[NB_H]
    isem = scr[NB_H + 1:2 * NB_H + 1]
    ssem = scr[2 * NB_H + 1:3 * NB_H + 1]
    acc = scr[-1]
    cid = lax.axis_index("c")
    sid = lax.axis_index("s")
    base = (cid * NS + sid) * (E // NW)
    nit = (E // NW) // SUB
    nring = nit - 1

    _fill(obuf, SUB, 0.0)
    _zero_acc_slice(obuf, SUB, acc, sid)
    plsc.subcore_barrier()
    _fill(obuf, SUB, 1.0)

    def off_of(i):
        return pl.multiple_of(base + i * SUB, 8)

    def fire_idx(b, i):
        pltpu.async_copy(row.at[pl.ds(off_of(i), SUB)], idx[b], isem[b])

    def wait_idx(b):
        pltpu.make_async_copy(row.at[pl.ds(base, SUB)], idx[b],
                              isem[b]).wait()

    def fire_sc(b):
        pltpu.async_copy(obuf, acc.at[idx[b]], ssem[b], add=True)

    def wait_sc(b):
        pltpu.make_async_copy(obuf, acc.at[idx[b]], ssem[b]).wait()

    for b in range(NB_H):
        fire_idx(b, b)

    def round_body(j, _):
        for b in range(NB_H):
            wait_idx(b)
            fire_sc(b)
        for b in range(NB_H):
            i_next = j * NB_H + b + NB_H

            @pl.when(i_next < nring)
            def _(b=b, i_next=i_next):
                wait_sc(b)
                fire_idx(b, i_next)
        return 0

    lax.fori_loop(0, nring // NB_H, round_body, 0)
    for b in range(NB_H):
        wait_sc(b)

    i = nit - 1
    pltpu.sync_copy(row.at[pl.ds(off_of(i), SUB)], idx[0])
    pltpu.sync_copy(obuf, acc.at[idx[0]], add=True)

    plsc.subcore_barrier()
    _acc_writeback(acc, out, cid, sid)


@jax.jit
def _sc_hist(row):
    scratch = (
        [pltpu.VMEM((SUB,), jnp.int32) for _ in range(NB_H)]
        + [pltpu.VMEM((SUB, D), F32)]
        + [pltpu.SemaphoreType.DMA for _ in range(2 * NB_H)]
        + [pltpu.VMEM_SHARED((N, D), F32)]
    )
    f = pl.kernel(
        _sc_hist_body,
        mesh=plsc.VectorSubcoreMesh(core_axis_name="c", subcore_axis_name="s"),
        out_type=[jax.ShapeDtypeStruct((NC, N, D), F32)],
        scratch_types=scratch,
    )
    return f(row)[0]


def _counts_body(bh_ref, od0_ref, od1_ref, ncnt_ref, ecnt_ref):
    @pl.when(pl.program_id(0) == 0)
    def _():
        ncnt_ref[...] = jnp.zeros_like(ncnt_ref)
        ecnt_ref[...] = jnp.zeros_like(ecnt_ref)

    bh = bh_ref[...]
    ncnt_ref[...] += _dot_t(bh, jnp.ones((BLK_N, D), F32))
    ecnt_ref[...] += _dot_t(bh, od0_ref[...] + od1_ref[...])


@jax.jit
def _tc_counts(bh, odf):
    return pl.pallas_call(
        _counts_body,
        grid=(NBN,),
        in_specs=[
            pl.BlockSpec((BLK_N, B), lambda i: (i, 0)),
            pl.BlockSpec((BLK_N, D), lambda i: (i, 0)),
            pl.BlockSpec((BLK_N, D), lambda i: (i + NBN, 0)),
        ],
        out_specs=[
            pl.BlockSpec((B, D), lambda i: (0, 0)),
            pl.BlockSpec((B, D), lambda i: (0, 0)),
        ],
        out_shape=[
            jax.ShapeDtypeStruct((B, D), F32),
            jax.ShapeDtypeStruct((B, D), F32),
        ],
    )(bh, odf, odf)


def _node_body(x_ref, agg_ref, rs_ref, bh_ref, u_ref,
               v1x_ref, v1a_ref, v1u_ref, c1_ref,
               v2_ref, c2_ref, v3_ref, c3_ref,
               xo_ref, nsum_ref, esum_ref):
    @pl.when(pl.program_id(0) == 0)
    def _():
        nsum_ref[...] = jnp.zeros_like(nsum_ref)
        esum_ref[...] = jnp.zeros_like(esum_ref)

    x = x_ref[...]
    bh = bh_ref[...]
    ub = _dot(u_ref[...], v1u_ref[...]) + c1_ref[...]
    h1 = jnp.maximum(_dot(x, v1x_ref[...]) + _dot(agg_ref[...], v1a_ref[...])
                     + _dot(bh, ub), 0.0)
    h2 = jnp.maximum(_dot(h1, v2_ref[...]) + c2_ref[...], 0.0)
    xr = _dot(h2, v3_ref[...]) + c3_ref[...]
    xo_ref[...] = x + xr
    nsum_ref[...] += _dot_t(bh, xr)
    esum_ref[...] += _dot_t(bh, rs_ref[...])


@jax.jit
def _tc_node(x, aggf, bh, u, v1x, v1a, v1u, c1, v2, c2, v3, c3):
    return pl.pallas_call(
        _node_body,
        grid=(NBN,),
        in_specs=[
            pl.BlockSpec((BLK_N, D), lambda i: (i, 0)),
            pl.BlockSpec((BLK_N, D), lambda i: (i, 0)),
            pl.BlockSpec((BLK_N, D), lambda i: (i + NBN, 0)),
            pl.BlockSpec((BLK_N, B), lambda i: (i, 0)),
            pl.BlockSpec((B, D), lambda i: (0, 0)),
            pl.BlockSpec((D, D), lambda i: (0, 0)),
            pl.BlockSpec((D, D), lambda i: (0, 0)),
            pl.BlockSpec((D, D), lambda i: (0, 0)),
            pl.BlockSpec((1, D), lambda i: (0, 0)),
            pl.BlockSpec((D, D), lambda i: (0, 0)),
            pl.BlockSpec((1, D), lambda i: (0, 0)),
            pl.BlockSpec((D, D), lambda i: (0, 0)),
            pl.BlockSpec((1, D), lambda i: (0, 0)),
        ],
        out_specs=[
            pl.BlockSpec((BLK_N, D), lambda i: (i, 0)),
            pl.BlockSpec((B, D), lambda i: (0, 0)),
            pl.BlockSpec((B, D), lambda i: (0, 0)),
        ],
        out_shape=[
            jax.ShapeDtypeStruct((N, D), F32),
            jax.ShapeDtypeStruct((B, D), F32),
            jax.ShapeDtypeStruct((B, D), F32),
        ],
    )(x, aggf, aggf, bh, u, v1x, v1a, v1u, c1, v2, c2, v3, c3)


def _glob_body(u_ref, nsum_ref, ncnt_ref, esum_ref, ecnt_ref,
               g1u_ref, g1n_ref, g1e_ref, g1b_ref,
               g2_ref, g2b_ref, g3_ref, g3b_ref, uo_ref):
    u = u_ref[...]
    nm = nsum_ref[...] / jnp.maximum(ncnt_ref[...], 1.0)
    em = esum_ref[...] / jnp.maximum(ecnt_ref[...], 1.0)
    h1 = jnp.maximum(_dot(u, g1u_ref[...]) + _dot(nm, g1n_ref[...])
                     + _dot(em, g1e_ref[...]) + g1b_ref[...], 0.0)
    h2 = jnp.maximum(_dot(h1, g2_ref[...]) + g2b_ref[...], 0.0)
    uo_ref[...] = u + _dot(h2, g3_ref[...]) + g3b_ref[...]


@jax.jit
def _tc_glob(u, nsum, ncnt, esum, ecnt, g1u, g1n, g1e, g1b, g2, g2b, g3, g3b):
    return pl.pallas_call(
        _glob_body,
        out_shape=jax.ShapeDtypeStruct((B, D), F32),
    )(u, nsum, ncnt, esum, ecnt, g1u, g1n, g1e, g1b, g2, g2b, g3, g3b)


def kernel(x, edge_index, edge_attr, u, batch, params):
    if u.ndim == 1:
        u = u[None]
    row = edge_index[0]
    col = edge_index[1]
    rc = jnp.concatenate([col, row])
    bh = (batch[:, None] == jnp.arange(B, dtype=batch.dtype)[None, :]
          ).astype(F32)

    odeg = _sc_hist(row)
    ncnt, ecnt = _tc_counts(bh, odeg.reshape(NC * N, D))

    for p in params:
        (w1, b1), (w2, b2), (w3, b3) = p['edge']
        (v1, c1), (v2, c2), (v3, c3) = p['node']
        (g1, g1b), (g2, g2b), (g3, g3b) = p['glob']
        w1s, w1d, w1e, w1u = w1[:D], w1[D:2 * D], w1[2 * D:3 * D], w1[3 * D:]
        v1x, v1a, v1u = v1[:D], v1[D:2 * D], v1[2 * D:]
        g1u, g1n, g1e = g1[:D], g1[D:2 * D], g1[2 * D:]

        xs2, xd = _tc_pre(x, bh, u, w1s, w1d, w1u, b1[None])
        gs = _sc_gather(xs2, xd, row, col)
        er, eo = _tc_edge(gs, edge_attr, w1e, w2, b2[None], w3, b3[None])
        aggrs = _sc_scatter(er, rc)
        xo, nsum, esum = _tc_node(x, aggrs.reshape(NC * N, D), bh, u,
                                  v1x, v1a, v1u, c1[None],
                                  v2, c2[None], v3, c3[None])
        uo = _tc_glob(u, nsum, ncnt, esum, ecnt, g1u, g1n, g1e, g1b[None],
                      g2, g2b[None], g3, g3b[None])
        x, edge_attr, u = xo, eo, uo

    return x, edge_attr, u
